# Initial kernel scaffold; baseline (speedup 1.0000x reference)
import jax
import jax.numpy as jnp
from jax.experimental import pallas as pl


def kernel(x, edge_index, W1, b1, W2, b2, W3, b3):
    raise NotImplementedError("write your pallas kernel here")


# speedup vs baseline: 5.1346x; 5.1346x over previous
import functools

import jax
import jax.numpy as jnp
from jax import lax
from jax.experimental import pallas as pl
from jax.experimental.pallas import tpu as pltpu
from jax.experimental.pallas import tpu_sc as plsc

NC = 2
NS = 16
NW = NC * NS
CHUNK = 128
RB = 512


def _make_prop(NP, EP, Wd):
  n_chunks = EP // CHUNK
  per_tile = n_chunks // NW
  rows_pt = NP // NS
  assert n_chunks % NW == 0 and NP % (NS * CHUNK) == 0
  mesh = plsc.VectorSubcoreMesh(core_axis_name="c", subcore_axis_name="s")

  @functools.partial(
      pl.kernel,
      out_type=jax.ShapeDtypeStruct((NC, NP, Wd), jnp.float32),
      mesh=mesh,
      scratch_types=[
          pltpu.VMEM((CHUNK,), jnp.int32),
          pltpu.VMEM((CHUNK,), jnp.int32),
          pltpu.VMEM((CHUNK, Wd), jnp.float32),
          pltpu.VMEM_SHARED((NP, Wd), jnp.float32),
          pltpu.SemaphoreType.DMA,
      ],
  )
  def prop(src_hbm, dst_hbm, h_hbm, out_hbm, idx_s, idx_d, rows, acc, sem):
    cid = lax.axis_index("c")
    sid = lax.axis_index("s")
    wid = cid * NS + sid

    def zbody(i, carry):
      for c in range(Wd // 16):
        rows[i, pl.ds(c * 16, 16)] = jnp.zeros((16,), jnp.float32)
      return carry
    lax.fori_loop(0, CHUNK, zbody, 0)
    for b in range(rows_pt // CHUNK):
      pltpu.sync_copy(rows, acc.at[pl.ds(sid * rows_pt + b * CHUNK, CHUNK)])
    plsc.subcore_barrier()

    def body(i, carry):
      g = pl.multiple_of((wid * per_tile + i) * CHUNK, CHUNK)
      pltpu.sync_copy(src_hbm.at[pl.ds(g, CHUNK)], idx_s)
      pltpu.sync_copy(dst_hbm.at[pl.ds(g, CHUNK)], idx_d)
      pltpu.async_copy(h_hbm.at[idx_s], rows, sem).wait()
      pltpu.sync_copy(rows, acc.at[idx_d], add=True)
      return carry
    lax.fori_loop(0, per_tile, body, 0)
    plsc.subcore_barrier()

    pltpu.sync_copy(acc.at[pl.ds(sid * rows_pt, rows_pt)],
                    out_hbm.at[cid, pl.ds(sid * rows_pt, rows_pt)])

  return prop


def _make_deg(NP, EP):
  WD = 128
  n_chunks = EP // CHUNK
  per_tile = n_chunks // NS
  rows_pt = NP // NS
  assert n_chunks % NS == 0 and NP % (NS * CHUNK) == 0
  mesh = plsc.VectorSubcoreMesh(core_axis_name="c", subcore_axis_name="s")

  @functools.partial(
      pl.kernel,
      out_type=jax.ShapeDtypeStruct((NC, NP, WD), jnp.float32),
      mesh=mesh,
      scratch_types=[
          pltpu.VMEM((CHUNK,), jnp.int32),
          pltpu.VMEM((CHUNK, WD), jnp.float32),
          pltpu.VMEM_SHARED((NP, WD), jnp.float32),
      ],
  )
  def deg(idx_hbm, out_hbm, idx_v, ones_v, acc):
    cid = lax.axis_index("c")
    sid = lax.axis_index("s")

    def fill(val):
      def body(i, carry):
        for c in range(WD // 16):
          ones_v[i, pl.ds(c * 16, 16)] = jnp.full((16,), val, jnp.float32)
        return carry
      lax.fori_loop(0, CHUNK, body, 0)

    fill(0.0)
    for b in range(rows_pt // CHUNK):
      pltpu.sync_copy(ones_v, acc.at[pl.ds(sid * rows_pt + b * CHUNK, CHUNK)])
    fill(1.0)
    plsc.subcore_barrier()

    def body(i, carry):
      g = pl.multiple_of((sid * per_tile + i) * CHUNK, CHUNK)
      pltpu.sync_copy(idx_hbm.at[cid, pl.ds(g, CHUNK)], idx_v)
      pltpu.sync_copy(ones_v, acc.at[idx_v], add=True)
      return carry
    lax.fori_loop(0, per_tile, body, 0)
    plsc.subcore_barrier()

    pltpu.sync_copy(acc.at[pl.ds(sid * rows_pt, rows_pt)],
                    out_hbm.at[cid, pl.ds(sid * rows_pt, rows_pt)])

  return deg


def _axpy_body(p_ref, h_ref, sa_ref, sb_ref, u_ref):
  s = p_ref[0] + p_ref[1]
  u_ref[...] = s * sa_ref[...] + h_ref[...] * sb_ref[...]


def _mm_body(t_ref, sc_ref, w_ref, b_ref, y_ref):
  agg = (t_ref[0] + t_ref[1]) * sc_ref[...]
  y = jnp.dot(agg, w_ref[...], preferred_element_type=jnp.float32,
              precision=lax.Precision.HIGHEST)
  y_ref[...] = jnp.maximum(y + b_ref[...], 0.0)


def _make_tc(NP, D):
  grid = (NP // RB,)
  axpy = pl.pallas_call(
      _axpy_body,
      grid=grid,
      in_specs=[
          pl.BlockSpec((NC, RB, D), lambda i: (0, i, 0)),
          pl.BlockSpec((RB, D), lambda i: (i, 0)),
          pl.BlockSpec((RB, 1), lambda i: (i, 0)),
          pl.BlockSpec((RB, 1), lambda i: (i, 0)),
      ],
      out_specs=pl.BlockSpec((RB, D), lambda i: (i, 0)),
      out_shape=jax.ShapeDtypeStruct((NP, D), jnp.float32),
  )
  mm = pl.pallas_call(
      _mm_body,
      grid=grid,
      in_specs=[
          pl.BlockSpec((NC, RB, D), lambda i: (0, i, 0)),
          pl.BlockSpec((RB, 1), lambda i: (i, 0)),
          pl.BlockSpec((D, D), lambda i: (0, 0)),
          pl.BlockSpec((1, D), lambda i: (0, 0)),
      ],
      out_specs=pl.BlockSpec((RB, D), lambda i: (i, 0)),
      out_shape=jax.ShapeDtypeStruct((NP, D), jnp.float32),
  )
  return axpy, mm


@jax.jit
def kernel(x, edge_index, W1, b1, W2, b2, W3, b3):
  N, D = x.shape
  E = edge_index.shape[1]

  NPQ = NS * CHUNK
  NP = ((N + NPQ) // NPQ) * NPQ
  EPQ = NW * CHUNK
  EP = ((E + EPQ - 1) // EPQ) * EPQ

  src = edge_index[0]
  dst = edge_index[1]
  npad = EP - E
  if npad:
    pad_idx = N + (jnp.arange(npad, dtype=jnp.int32) % (NP - N))
    src = jnp.concatenate([src, pad_idx])
    dst = jnp.concatenate([dst, pad_idx])

  h0 = jnp.pad(x, ((0, NP - N), (0, 0)))

  deg = _make_deg(NP, EP)
  propD = _make_prop(NP, EP, D)
  axpy, mm = _make_tc(NP, D)

  degs = deg(jnp.stack([dst, src]))
  in_deg = degs[0, :, 0:1]
  out_deg = degs[1, :, 0:1]

  ic = jnp.maximum(in_deg, 1.0)
  oc = jnp.maximum(out_deg, 1.0)
  ors = oc ** -0.5
  pos = in_deg > 0
  sa = jnp.where(pos, 1.0 / ic, 0.0) * ors
  sb = jnp.where(pos, 0.0, 1.0) * ors
  sc_v = ic ** -0.5

  h = h0
  for (W, b) in ((W1, b1), (W2, b2), (W3, b3)):
    p = propD(src, dst, h)
    u = axpy(p, h, sa, sb)
    t = propD(src, dst, u)
    h = mm(t, sc_v, W, b.reshape(1, D))
  return h[:N]


# speedup vs baseline: 7.8794x; 1.5346x over previous
import functools

import jax
import jax.numpy as jnp
from jax import lax
from jax.experimental import pallas as pl
from jax.experimental.pallas import tpu as pltpu
from jax.experimental.pallas import tpu_sc as plsc

NC = 2
NS = 16
NW = NC * NS
CHUNK = 128
NBUF = 2
RB = 512


def _make_prop(NP, EP, Wd):
  n_chunks = EP // CHUNK
  per_tile = n_chunks // NW
  rows_pt = NP // NS
  assert n_chunks % (NW * NBUF) == 0 and NP % (NS * CHUNK) == 0
  mesh = plsc.VectorSubcoreMesh(core_axis_name="c", subcore_axis_name="s")

  scratch = ([pltpu.VMEM((CHUNK,), jnp.int32)] * NBUF +
             [pltpu.VMEM((CHUNK,), jnp.int32)] * NBUF +
             [pltpu.VMEM((CHUNK, Wd), jnp.float32)] * NBUF +
             [pltpu.VMEM_SHARED((NP, Wd), jnp.float32)] +
             [pltpu.SemaphoreType.DMA] * (2 * NBUF))

  @functools.partial(
      pl.kernel,
      out_type=jax.ShapeDtypeStruct((NC, NP, Wd), jnp.float32),
      mesh=mesh,
      scratch_types=scratch,
  )
  def prop(src_hbm, dst_hbm, h_hbm, out_hbm, *scr):
    idx_s = scr[0:NBUF]
    idx_d = scr[NBUF:2 * NBUF]
    rows = scr[2 * NBUF:3 * NBUF]
    acc = scr[3 * NBUF]
    gsem = scr[3 * NBUF + 1:3 * NBUF + 1 + NBUF]
    ssem = scr[3 * NBUF + 1 + NBUF:]
    cid = lax.axis_index("c")
    sid = lax.axis_index("s")
    wid = cid * NS + sid

    def zbody(i, carry):
      for c in range(Wd // 16):
        rows[0][i, pl.ds(c * 16, 16)] = jnp.zeros((16,), jnp.float32)
      return carry
    lax.fori_loop(0, CHUNK, zbody, 0)
    for b in range(rows_pt // CHUNK):
      pltpu.sync_copy(rows[0], acc.at[pl.ds(sid * rows_pt + b * CHUNK, CHUNK)])
    plsc.subcore_barrier()

    base = wid * per_tile
    def body(k, carry):
      for j in range(NBUF):
        g = pl.multiple_of((base + k * NBUF + j) * CHUNK, CHUNK)

        @pl.when(k > 0)
        def _wait_prev():
          pltpu.make_async_copy(rows[j], acc.at[idx_d[j]], ssem[j]).wait()

        pltpu.sync_copy(src_hbm.at[pl.ds(g, CHUNK)], idx_s[j])
        pltpu.sync_copy(dst_hbm.at[pl.ds(g, CHUNK)], idx_d[j])
        pltpu.async_copy(h_hbm.at[idx_s[j]], rows[j], gsem[j])
      for j in range(NBUF):
        pltpu.make_async_copy(h_hbm.at[idx_s[j]], rows[j], gsem[j]).wait()
        pltpu.async_copy(rows[j], acc.at[idx_d[j]], ssem[j], add=True)
      return carry
    lax.fori_loop(0, per_tile // NBUF, body, 0)
    for j in range(NBUF):
      pltpu.make_async_copy(rows[j], acc.at[idx_d[j]], ssem[j]).wait()
    plsc.subcore_barrier()

    pltpu.sync_copy(acc.at[pl.ds(sid * rows_pt, rows_pt)],
                    out_hbm.at[cid, pl.ds(sid * rows_pt, rows_pt)])

  return prop


def _make_deg(NP, EP):
  WD = 128
  n_chunks = EP // CHUNK
  per_tile = n_chunks // NS
  rows_pt = NP // NS
  assert n_chunks % (NS * NBUF) == 0 and NP % (NS * CHUNK) == 0
  mesh = plsc.VectorSubcoreMesh(core_axis_name="c", subcore_axis_name="s")

  scratch = ([pltpu.VMEM((CHUNK,), jnp.int32)] * NBUF +
             [pltpu.VMEM((CHUNK, WD), jnp.float32)] +
             [pltpu.VMEM_SHARED((NP, WD), jnp.float32)] +
             [pltpu.SemaphoreType.DMA] * NBUF)

  @functools.partial(
      pl.kernel,
      out_type=jax.ShapeDtypeStruct((NC, NP, WD), jnp.float32),
      mesh=mesh,
      scratch_types=scratch,
  )
  def deg(idx_hbm, out_hbm, *scr):
    idx_v = scr[0:NBUF]
    ones_v = scr[NBUF]
    acc = scr[NBUF + 1]
    ssem = scr[NBUF + 2:]
    cid = lax.axis_index("c")
    sid = lax.axis_index("s")

    def fill(val):
      def body(i, carry):
        for c in range(WD // 16):
          ones_v[i, pl.ds(c * 16, 16)] = jnp.full((16,), val, jnp.float32)
        return carry
      lax.fori_loop(0, CHUNK, body, 0)

    fill(0.0)
    for b in range(rows_pt // CHUNK):
      pltpu.sync_copy(ones_v, acc.at[pl.ds(sid * rows_pt + b * CHUNK, CHUNK)])
    fill(1.0)
    plsc.subcore_barrier()

    base = sid * per_tile
    def body(k, carry):
      for j in range(NBUF):
        g = pl.multiple_of((base + k * NBUF + j) * CHUNK, CHUNK)

        @pl.when(k > 0)
        def _wait_prev():
          pltpu.make_async_copy(ones_v, acc.at[idx_v[j]], ssem[j]).wait()

        pltpu.sync_copy(idx_hbm.at[cid, pl.ds(g, CHUNK)], idx_v[j])
        pltpu.async_copy(ones_v, acc.at[idx_v[j]], ssem[j], add=True)
      return carry
    lax.fori_loop(0, per_tile // NBUF, body, 0)
    for j in range(NBUF):
      pltpu.make_async_copy(ones_v, acc.at[idx_v[j]], ssem[j]).wait()
    plsc.subcore_barrier()

    pltpu.sync_copy(acc.at[pl.ds(sid * rows_pt, rows_pt)],
                    out_hbm.at[cid, pl.ds(sid * rows_pt, rows_pt)])

  return deg


def _axpy_body(p_ref, h_ref, sa_ref, sb_ref, u_ref):
  s = p_ref[0] + p_ref[1]
  u_ref[...] = s * sa_ref[...] + h_ref[...] * sb_ref[...]


def _mm_body(t_ref, sc_ref, w_ref, b_ref, y_ref):
  agg = (t_ref[0] + t_ref[1]) * sc_ref[...]
  y = jnp.dot(agg, w_ref[...], preferred_element_type=jnp.float32,
              precision=lax.Precision.HIGHEST)
  y_ref[...] = jnp.maximum(y + b_ref[...], 0.0)


def _make_tc(NP, D):
  grid = (NP // RB,)
  axpy = pl.pallas_call(
      _axpy_body,
      grid=grid,
      in_specs=[
          pl.BlockSpec((NC, RB, D), lambda i: (0, i, 0)),
          pl.BlockSpec((RB, D), lambda i: (i, 0)),
          pl.BlockSpec((RB, 1), lambda i: (i, 0)),
          pl.BlockSpec((RB, 1), lambda i: (i, 0)),
      ],
      out_specs=pl.BlockSpec((RB, D), lambda i: (i, 0)),
      out_shape=jax.ShapeDtypeStruct((NP, D), jnp.float32),
  )
  mm = pl.pallas_call(
      _mm_body,
      grid=grid,
      in_specs=[
          pl.BlockSpec((NC, RB, D), lambda i: (0, i, 0)),
          pl.BlockSpec((RB, 1), lambda i: (i, 0)),
          pl.BlockSpec((D, D), lambda i: (0, 0)),
          pl.BlockSpec((1, D), lambda i: (0, 0)),
      ],
      out_specs=pl.BlockSpec((RB, D), lambda i: (i, 0)),
      out_shape=jax.ShapeDtypeStruct((NP, D), jnp.float32),
  )
  return axpy, mm


@jax.jit
def kernel(x, edge_index, W1, b1, W2, b2, W3, b3):
  N, D = x.shape
  E = edge_index.shape[1]

  NPQ = NS * CHUNK
  NP = ((N + NPQ) // NPQ) * NPQ
  EPQ = NW * CHUNK * NBUF
  EP = ((E + EPQ - 1) // EPQ) * EPQ

  src = edge_index[0]
  dst = edge_index[1]
  npad = EP - E
  if npad:
    pad_idx = N + (jnp.arange(npad, dtype=jnp.int32) % (NP - N))
    src = jnp.concatenate([src, pad_idx])
    dst = jnp.concatenate([dst, pad_idx])

  h0 = jnp.pad(x, ((0, NP - N), (0, 0)))

  deg = _make_deg(NP, EP)
  propD = _make_prop(NP, EP, D)
  axpy, mm = _make_tc(NP, D)

  degs = deg(jnp.stack([dst, src]))
  in_deg = degs[0, :, 0:1]
  out_deg = degs[1, :, 0:1]

  ic = jnp.maximum(in_deg, 1.0)
  oc = jnp.maximum(out_deg, 1.0)
  ors = oc ** -0.5
  pos = in_deg > 0
  sa = jnp.where(pos, 1.0 / ic, 0.0) * ors
  sb = jnp.where(pos, 0.0, 1.0) * ors
  sc_v = ic ** -0.5

  h = h0
  for (W, b) in ((W1, b1), (W2, b2), (W3, b3)):
    p = propD(src, dst, h)
    u = axpy(p, h, sa, sb)
    t = propD(src, dst, u)
    h = mm(t, sc_v, W, b.reshape(1, D))
  return h[:N]


# speedup vs baseline: 9.7321x; 1.2351x over previous
import functools

import jax
import jax.numpy as jnp
from jax import lax
from jax.experimental import pallas as pl
from jax.experimental.pallas import tpu as pltpu
from jax.experimental.pallas import tpu_sc as plsc

NC = 2
NS = 16
NW = NC * NS
CHUNK = 128
NBUF = 2
RB = 512


def _make_prop(NP, EP, Wd):
  NG = 2
  SUP = 8
  n_chunks = EP // CHUNK
  per_tile = n_chunks // NW
  rows_pt = NP // NS
  assert n_chunks % (NW * 2 * SUP) == 0 and NP % (NS * CHUNK) == 0
  mesh = plsc.VectorSubcoreMesh(core_axis_name="c", subcore_axis_name="s")

  scratch = ([pltpu.VMEM((SUP, CHUNK), jnp.int32)] * 4 +
             [pltpu.VMEM((CHUNK, Wd), jnp.float32)] * NG +
             [pltpu.VMEM_SHARED((NP, Wd), jnp.float32)] +
             [pltpu.SemaphoreType.DMA] * (2 * NG))

  @functools.partial(
      pl.kernel,
      out_type=jax.ShapeDtypeStruct((NC, NP, Wd), jnp.float32),
      mesh=mesh,
      scratch_types=scratch,
  )
  def prop(src_hbm, dst_hbm, h_hbm, out_hbm, *scr):
    isrc = scr[0:2]
    idst = scr[2:4]
    rows = scr[4:4 + NG]
    acc = scr[4 + NG]
    gsem = scr[5 + NG:5 + 2 * NG]
    ssem = scr[5 + 2 * NG:]
    cid = lax.axis_index("c")
    sid = lax.axis_index("s")
    wid = cid * NS + sid
    base = wid * per_tile

    def zbody(i, carry):
      for c in range(Wd // 16):
        rows[0][i, pl.ds(c * 16, 16)] = jnp.zeros((16,), jnp.float32)
      return carry
    lax.fori_loop(0, CHUNK, zbody, 0)
    for b in range(rows_pt // CHUNK):
      pltpu.sync_copy(rows[0], acc.at[pl.ds(sid * rows_pt + b * CHUNK, CHUNK)])
    plsc.subcore_barrier()

    def gdesc(b, i, j):
      return pltpu.make_async_copy(h_hbm.at[isrc[b].at[i]], rows[j], gsem[j])

    def sdesc(b, i, j):
      return pltpu.make_async_copy(rows[j], acc.at[idst[b].at[i]], ssem[j])

    def body(k, carry):
      for b in range(2):
        blk = pl.multiple_of(base + (2 * k + b) * SUP, SUP)
        pltpu.sync_copy(src_hbm.at[pl.ds(blk, SUP)], isrc[b])
        pltpu.sync_copy(dst_hbm.at[pl.ds(blk, SUP)], idst[b])
        for i in range(SUP):
          j = i % 2
          if b == 0 and i < 2:
            @pl.when(k > 0)
            def _w():
              sdesc(1, SUP - 2 + i, j).wait()
          else:
            sdesc(b, i - 2, j).wait() if i >= 2 else sdesc(b ^ 1, SUP - 2 + i, j).wait()
          gdesc(b, i, j).start()
          if b == 0 and i == 0:
            @pl.when(k > 0)
            def _d():
              gdesc(1, SUP - 1, j ^ 1).wait()
              sdesc(1, SUP - 1, j ^ 1).start(add=True)
          elif i == 0:
            gdesc(0, SUP - 1, j ^ 1).wait()
            sdesc(0, SUP - 1, j ^ 1).start(add=True)
          else:
            gdesc(b, i - 1, j ^ 1).wait()
            sdesc(b, i - 1, j ^ 1).start(add=True)
      return carry

    lax.fori_loop(0, per_tile // (2 * SUP), body, 0)
    gdesc(1, SUP - 1, (SUP - 1) % 2).wait()
    sdesc(1, SUP - 1, (SUP - 1) % 2).start(add=True)
    sdesc(1, SUP - 2, SUP % 2).wait()
    sdesc(1, SUP - 1, (SUP - 1) % 2).wait()
    plsc.subcore_barrier()

    pltpu.sync_copy(acc.at[pl.ds(sid * rows_pt, rows_pt)],
                    out_hbm.at[cid, pl.ds(sid * rows_pt, rows_pt)])

  return prop


def _make_deg(NP, EP):
  WD = 128
  NSC = 2
  n_chunks = EP // CHUNK
  per_tile = n_chunks // NS
  rows_pt = NP // NS
  assert n_chunks % (NS * NSC) == 0 and NP % (NS * CHUNK) == 0
  mesh = plsc.VectorSubcoreMesh(core_axis_name="c", subcore_axis_name="s")

  scratch = ([pltpu.VMEM((per_tile, CHUNK), jnp.int32)] +
             [pltpu.VMEM((CHUNK, WD), jnp.float32)] +
             [pltpu.VMEM_SHARED((NP, WD), jnp.float32)] +
             [pltpu.SemaphoreType.DMA] * NSC)

  @functools.partial(
      pl.kernel,
      out_type=jax.ShapeDtypeStruct((NC, NP, WD), jnp.float32),
      mesh=mesh,
      scratch_types=scratch,
  )
  def deg(idx_hbm, out_hbm, *scr):
    idx_all, ones_v, acc = scr[0], scr[1], scr[2]
    ssem = scr[3:]
    cid = lax.axis_index("c")
    sid = lax.axis_index("s")
    base = sid * per_tile
    pltpu.sync_copy(idx_hbm.at[cid, pl.ds(base, per_tile)], idx_all)

    def fill(val):
      def body(i, carry):
        for c in range(WD // 16):
          ones_v[i, pl.ds(c * 16, 16)] = jnp.full((16,), val, jnp.float32)
        return carry
      lax.fori_loop(0, CHUNK, body, 0)

    fill(0.0)
    for b in range(rows_pt // CHUNK):
      pltpu.sync_copy(ones_v, acc.at[pl.ds(sid * rows_pt + b * CHUNK, CHUNK)])
    fill(1.0)
    plsc.subcore_barrier()

    def sdesc(c, p):
      return pltpu.make_async_copy(ones_v, acc.at[idx_all.at[c]], ssem[p])

    def body(k, carry):
      for j in range(NSC):
        c = k * NSC + j

        @pl.when(k > 0)
        def _wait_prev():
          sdesc(c - NSC, j).wait()

        sdesc(c, j).start(add=True)
      return carry
    lax.fori_loop(0, per_tile // NSC, body, 0)
    for t in range(NSC):
      cw = per_tile - NSC + t
      sdesc(cw, cw % NSC).wait()
    plsc.subcore_barrier()

    pltpu.sync_copy(acc.at[pl.ds(sid * rows_pt, rows_pt)],
                    out_hbm.at[cid, pl.ds(sid * rows_pt, rows_pt)])

  return deg


def _axpy_body(p_ref, h_ref, sa_ref, sb_ref, u_ref):
  s = p_ref[0] + p_ref[1]
  u_ref[...] = s * sa_ref[...] + h_ref[...] * sb_ref[...]


def _mm_body(t_ref, sc_ref, w_ref, b_ref, y_ref):
  agg = (t_ref[0] + t_ref[1]) * sc_ref[...]
  y = jnp.dot(agg, w_ref[...], preferred_element_type=jnp.float32,
              precision=lax.Precision.HIGHEST)
  y_ref[...] = jnp.maximum(y + b_ref[...], 0.0)


def _make_tc(NP, D):
  grid = (NP // RB,)
  axpy = pl.pallas_call(
      _axpy_body,
      grid=grid,
      in_specs=[
          pl.BlockSpec((NC, RB, D), lambda i: (0, i, 0)),
          pl.BlockSpec((RB, D), lambda i: (i, 0)),
          pl.BlockSpec((RB, 1), lambda i: (i, 0)),
          pl.BlockSpec((RB, 1), lambda i: (i, 0)),
      ],
      out_specs=pl.BlockSpec((RB, D), lambda i: (i, 0)),
      out_shape=jax.ShapeDtypeStruct((NP, D), jnp.float32),
  )
  mm = pl.pallas_call(
      _mm_body,
      grid=grid,
      in_specs=[
          pl.BlockSpec((NC, RB, D), lambda i: (0, i, 0)),
          pl.BlockSpec((RB, 1), lambda i: (i, 0)),
          pl.BlockSpec((D, D), lambda i: (0, 0)),
          pl.BlockSpec((1, D), lambda i: (0, 0)),
      ],
      out_specs=pl.BlockSpec((RB, D), lambda i: (i, 0)),
      out_shape=jax.ShapeDtypeStruct((NP, D), jnp.float32),
  )
  return axpy, mm


@jax.jit
def kernel(x, edge_index, W1, b1, W2, b2, W3, b3):
  N, D = x.shape
  E = edge_index.shape[1]

  NPQ = NS * CHUNK
  NP = ((N + NPQ) // NPQ) * NPQ
  EPQ = NW * CHUNK * 16
  EP = ((E + EPQ - 1) // EPQ) * EPQ

  src = edge_index[0]
  dst = edge_index[1]
  npad = EP - E
  if npad:
    pad_idx = N + (jnp.arange(npad, dtype=jnp.int32) % (NP - N))
    src = jnp.concatenate([src, pad_idx])
    dst = jnp.concatenate([dst, pad_idx])

  h0 = jnp.pad(x, ((0, NP - N), (0, 0)))

  n_chunks = EP // CHUNK
  src_r = src.reshape(n_chunks, CHUNK)
  dst_r = dst.reshape(n_chunks, CHUNK)

  deg = _make_deg(NP, EP)
  propD = _make_prop(NP, EP, D)
  axpy, mm = _make_tc(NP, D)

  degs = deg(jnp.stack([dst_r, src_r]))
  h0 = jnp.where(jnp.isfinite(degs[0, 0, 0]), h0, 0.0)
  in_deg = degs[0, :, 0:1]
  out_deg = degs[1, :, 0:1]

  ic = jnp.maximum(in_deg, 1.0)
  oc = jnp.maximum(out_deg, 1.0)
  ors = oc ** -0.5
  pos = in_deg > 0
  sa = jnp.where(pos, 1.0 / ic, 0.0) * ors
  sb = jnp.where(pos, 0.0, 1.0) * ors
  sc_v = ic ** -0.5

  h = h0
  for (W, b) in ((W1, b1), (W2, b2), (W3, b3)):
    p = propD(src_r, dst_r, h)
    u = axpy(p, h, sa, sb)
    t = propD(src_r, dst_r, u)
    h = mm(t, sc_v, W, b.reshape(1, D))
  return h[:N]


# speedup vs baseline: 10.4671x; 1.0755x over previous
import functools

import jax
import jax.numpy as jnp
from jax import lax
from jax.experimental import pallas as pl
from jax.experimental.pallas import tpu as pltpu
from jax.experimental.pallas import tpu_sc as plsc

NC = 2
NS = 16
NW = NC * NS
CHUNK = 128
NBUF = 2
RB = 512


def _make_prop(NP, EP, Wd):
  CK = 64
  NG = 4
  DL = 3
  SUP = 16
  n_chunks = EP // CK
  per_tile = n_chunks // NW
  rows_pt = NP // NS
  assert n_chunks % (NW * 2 * SUP) == 0 and NP % (NS * CK) == 0
  assert SUP % NG == 0 and NG > DL
  mesh = plsc.VectorSubcoreMesh(core_axis_name="c", subcore_axis_name="s")

  scratch = ([pltpu.VMEM((SUP, CK), jnp.int32)] * 4 +
             [pltpu.VMEM((CK, Wd), jnp.float32)] * NG +
             [pltpu.VMEM_SHARED((NP, Wd), jnp.float32)] +
             [pltpu.SemaphoreType.DMA] * (2 * NG))

  @functools.partial(
      pl.kernel,
      out_type=jax.ShapeDtypeStruct((NC, NP, Wd), jnp.float32),
      mesh=mesh,
      scratch_types=scratch,
  )
  def prop(src_hbm, dst_hbm, h_hbm, out_hbm, *scr):
    isrc = scr[0:2]
    idst = scr[2:4]
    rows = scr[4:4 + NG]
    acc = scr[4 + NG]
    gsem = scr[5 + NG:5 + 2 * NG]
    ssem = scr[5 + 2 * NG:5 + 3 * NG]
    cid = lax.axis_index("c")
    sid = lax.axis_index("s")
    wid = cid * NS + sid
    base = wid * per_tile

    def zbody(i, carry):
      for c in range(Wd // 16):
        rows[0][i, pl.ds(c * 16, 16)] = jnp.zeros((16,), jnp.float32)
        rows[1][i, pl.ds(c * 16, 16)] = jnp.zeros((16,), jnp.float32)
      return carry
    lax.fori_loop(0, CK, zbody, 0)
    for b in range(rows_pt // (2 * CK)):
      pltpu.sync_copy(rows[0], acc.at[pl.ds(sid * rows_pt + 2 * b * CK, CK)])
      pltpu.sync_copy(rows[1], acc.at[pl.ds(sid * rows_pt + (2 * b + 1) * CK, CK)])
    plsc.subcore_barrier()

    def gdesc(b, i, j):
      return pltpu.make_async_copy(h_hbm.at[isrc[b].at[i]], rows[j], gsem[j])

    def sdesc(b, i, j):
      return pltpu.make_async_copy(rows[j], acc.at[idst[b].at[i]], ssem[j])

    def body(k, carry):
      for b in range(2):
        blk = pl.multiple_of(base + (2 * k + b) * SUP, SUP)
        pltpu.sync_copy(src_hbm.at[pl.ds(blk, SUP)], isrc[b])
        pltpu.sync_copy(dst_hbm.at[pl.ds(blk, SUP)], idst[b])
        for i in range(SUP):
          j = i % NG
          bw, iw = (b, i - NG) if i >= NG else (b ^ 1, SUP - NG + i)
          if b == 0 and i < NG:
            @pl.when(k > 0)
            def _w(bw=bw, iw=iw, j=j):
              sdesc(bw, iw, j).wait()
          else:
            sdesc(bw, iw, j).wait()
          gdesc(b, i, j).start()
          jd = (j - DL) % NG
          bd, idd = (b, i - DL) if i >= DL else (b ^ 1, SUP - DL + i)
          if b == 0 and i < DL:
            @pl.when(k > 0)
            def _d(bd=bd, idd=idd, jd=jd):
              gdesc(bd, idd, jd).wait()
              sdesc(bd, idd, jd).start(add=True)
          else:
            gdesc(bd, idd, jd).wait()
            sdesc(bd, idd, jd).start(add=True)
      return carry

    lax.fori_loop(0, per_tile // (2 * SUP), body, 0)
    for t in range(DL):
      i = SUP - DL + t
      gdesc(1, i, i % NG).wait()
      sdesc(1, i, i % NG).start(add=True)
    for t in range(NG):
      i = SUP - NG + t
      sdesc(1, i, i % NG).wait()
    plsc.subcore_barrier()

    pltpu.sync_copy(acc.at[pl.ds(sid * rows_pt, rows_pt)],
                    out_hbm.at[cid, pl.ds(sid * rows_pt, rows_pt)])

  return prop


def _make_deg(NP, EP):
  WD = 128
  NSC = 2
  n_chunks = EP // CHUNK
  per_tile = n_chunks // NS
  rows_pt = NP // NS
  assert n_chunks % (NS * NSC) == 0 and NP % (NS * CHUNK) == 0
  mesh = plsc.VectorSubcoreMesh(core_axis_name="c", subcore_axis_name="s")

  scratch = ([pltpu.VMEM((per_tile, CHUNK), jnp.int32)] +
             [pltpu.VMEM((CHUNK, WD), jnp.float32)] +
             [pltpu.VMEM_SHARED((NP, WD), jnp.float32)] +
             [pltpu.SemaphoreType.DMA] * NSC)

  @functools.partial(
      pl.kernel,
      out_type=jax.ShapeDtypeStruct((NC, NP, WD), jnp.float32),
      mesh=mesh,
      scratch_types=scratch,
  )
  def deg(idx_hbm, out_hbm, *scr):
    idx_all, ones_v, acc = scr[0], scr[1], scr[2]
    ssem = scr[3:]
    cid = lax.axis_index("c")
    sid = lax.axis_index("s")
    base = sid * per_tile
    pltpu.sync_copy(idx_hbm.at[cid, pl.ds(base, per_tile)], idx_all)

    def fill(val):
      def body(i, carry):
        for c in range(WD // 16):
          ones_v[i, pl.ds(c * 16, 16)] = jnp.full((16,), val, jnp.float32)
        return carry
      lax.fori_loop(0, CHUNK, body, 0)

    fill(0.0)
    for b in range(rows_pt // CHUNK):
      pltpu.sync_copy(ones_v, acc.at[pl.ds(sid * rows_pt + b * CHUNK, CHUNK)])
    fill(1.0)
    plsc.subcore_barrier()

    def sdesc(c, p):
      return pltpu.make_async_copy(ones_v, acc.at[idx_all.at[c]], ssem[p])

    def body(k, carry):
      for j in range(NSC):
        c = k * NSC + j

        @pl.when(k > 0)
        def _wait_prev():
          sdesc(c - NSC, j).wait()

        sdesc(c, j).start(add=True)
      return carry
    lax.fori_loop(0, per_tile // NSC, body, 0)
    for t in range(NSC):
      cw = per_tile - NSC + t
      sdesc(cw, cw % NSC).wait()
    plsc.subcore_barrier()

    pltpu.sync_copy(acc.at[pl.ds(sid * rows_pt, rows_pt)],
                    out_hbm.at[cid, pl.ds(sid * rows_pt, rows_pt)])

  return deg


def _axpy_body(p_ref, h_ref, sa_ref, sb_ref, u_ref):
  s = p_ref[0] + p_ref[1]
  u_ref[...] = s * sa_ref[...] + h_ref[...] * sb_ref[...]


def _mm_body(t_ref, sc_ref, w_ref, b_ref, y_ref):
  agg = (t_ref[0] + t_ref[1]) * sc_ref[...]
  y = jnp.dot(agg, w_ref[...], preferred_element_type=jnp.float32,
              precision=lax.Precision.HIGHEST)
  y_ref[...] = jnp.maximum(y + b_ref[...], 0.0)


def _make_tc(NP, D):
  grid = (NP // RB,)
  axpy = pl.pallas_call(
      _axpy_body,
      grid=grid,
      in_specs=[
          pl.BlockSpec((NC, RB, D), lambda i: (0, i, 0)),
          pl.BlockSpec((RB, D), lambda i: (i, 0)),
          pl.BlockSpec((RB, 1), lambda i: (i, 0)),
          pl.BlockSpec((RB, 1), lambda i: (i, 0)),
      ],
      out_specs=pl.BlockSpec((RB, D), lambda i: (i, 0)),
      out_shape=jax.ShapeDtypeStruct((NP, D), jnp.float32),
  )
  mm = pl.pallas_call(
      _mm_body,
      grid=grid,
      in_specs=[
          pl.BlockSpec((NC, RB, D), lambda i: (0, i, 0)),
          pl.BlockSpec((RB, 1), lambda i: (i, 0)),
          pl.BlockSpec((D, D), lambda i: (0, 0)),
          pl.BlockSpec((1, D), lambda i: (0, 0)),
      ],
      out_specs=pl.BlockSpec((RB, D), lambda i: (i, 0)),
      out_shape=jax.ShapeDtypeStruct((NP, D), jnp.float32),
  )
  return axpy, mm


@jax.jit
def kernel(x, edge_index, W1, b1, W2, b2, W3, b3):
  N, D = x.shape
  E = edge_index.shape[1]

  NPQ = NS * CHUNK
  NP = ((N + NPQ) // NPQ) * NPQ
  EPQ = NW * CHUNK * 16
  EP = ((E + EPQ - 1) // EPQ) * EPQ

  src = edge_index[0]
  dst = edge_index[1]
  npad = EP - E
  if npad:
    pad_idx = N + (jnp.arange(npad, dtype=jnp.int32) % (NP - N))
    src = jnp.concatenate([src, pad_idx])
    dst = jnp.concatenate([dst, pad_idx])

  h0 = jnp.pad(x, ((0, NP - N), (0, 0)))

  n_chunks = EP // CHUNK
  src_r = src.reshape(n_chunks, CHUNK)
  dst_r = dst.reshape(n_chunks, CHUNK)
  src_p = src.reshape(EP // 64, 64)
  dst_p = dst.reshape(EP // 64, 64)

  deg = _make_deg(NP, EP)
  propD = _make_prop(NP, EP, D)
  axpy, mm = _make_tc(NP, D)

  degs = deg(jnp.stack([dst_r, src_r]))
  h0 = jnp.where(jnp.isfinite(degs[0, 0, 0]), h0, 0.0)
  in_deg = degs[0, :, 0:1]
  out_deg = degs[1, :, 0:1]

  ic = jnp.maximum(in_deg, 1.0)
  oc = jnp.maximum(out_deg, 1.0)
  ors = oc ** -0.5
  pos = in_deg > 0
  sa = jnp.where(pos, 1.0 / ic, 0.0) * ors
  sb = jnp.where(pos, 0.0, 1.0) * ors
  sc_v = ic ** -0.5

  h = h0
  for (W, b) in ((W1, b1), (W2, b2), (W3, b3)):
    p = propD(src_p, dst_p, h)
    u = axpy(p, h, sa, sb)
    t = propD(src_p, dst_p, u)
    h = mm(t, sc_v, W, b.reshape(1, D))
  return h[:N]
